# Initial kernel scaffold; baseline (speedup 1.0000x reference)
import functools

import jax
import jax.numpy as jnp
from jax import lax
from jax.experimental import pallas as pl
from jax.experimental.pallas import tpu as pltpu
from jax.experimental.pallas import tpu_sc as plsc

EPS = 1e-5

NC = 2
NS = 16
L = 16
NW = NC * NS

K = 80


def _sc_spmm_partials(feat, src, dst, w, zeros):
    n, d = feat.shape
    e = src.shape[0]
    ept = e // NW
    nchunk = ept // K
    rpt = n // NS

    mesh = plsc.VectorSubcoreMesh(core_axis_name="c", subcore_axis_name="s")

    @functools.partial(
        pl.kernel,
        mesh=mesh,
        out_type=jax.ShapeDtypeStruct((NC, n, d), jnp.float32),
        scratch_types=[
            pltpu.VMEM((K,), jnp.int32),
            pltpu.VMEM((K,), jnp.int32),
            pltpu.VMEM((K,), jnp.float32),
            pltpu.VMEM((K, d), jnp.float32),
            pltpu.VMEM_SHARED((n, d), jnp.float32),
            pltpu.SemaphoreType.DMA,
        ],
    )
    def k(feat_h, src_h, dst_h, w_h, zeros_h, out_h,
          src_v, dst_v, w_v, rows_v, acc, sem):
        c = lax.axis_index("c")
        s = lax.axis_index("s")
        wid = s * NC + c

        pltpu.sync_copy(zeros_h.at[pl.ds(s * rpt, rpt)],
                        acc.at[pl.ds(s * rpt, rpt)])
        plsc.subcore_barrier()

        def body(t, carry):
            base = wid * ept + t * K
            pltpu.sync_copy(src_h.at[pl.ds(base, K)], src_v)
            pltpu.sync_copy(dst_h.at[pl.ds(base, K)], dst_v)
            pltpu.sync_copy(w_h.at[pl.ds(base, K)], w_v)
            pltpu.async_copy(feat_h.at[src_v], rows_v, sem).wait()
            for i in range(K):
                ws = w_v[i]
                for j in range(d // L):
                    rows_v[i, pl.ds(j * L, L)] = rows_v[i, pl.ds(j * L, L)] * ws
            pltpu.sync_copy(rows_v, acc.at[dst_v], add=True)
            return carry

        lax.fori_loop(0, nchunk, body, 0)
        plsc.subcore_barrier()

        pltpu.sync_copy(acc.at[pl.ds(s * rpt, rpt)],
                        out_h.at[c, pl.ds(s * rpt, rpt)])

    return k(feat, src, dst, w, zeros)


def _tc_epilogue(feat, part0, part1, W0, b0, W1, b1, gamma, beta):
    n, d_out = feat.shape[0], W0.shape[1]

    def body(feat_r, p0_r, p1_r, w0_r, b0_r, w1_r, b1_r, g_r, be_r, out_r):
        hop1 = p0_r[...] + p1_r[...]
        p0 = jnp.maximum(
            jnp.dot(feat_r[...], w0_r[...], preferred_element_type=jnp.float32),
            0.0) + b0_r[...]
        p1 = jnp.maximum(
            jnp.dot(hop1, w1_r[...], preferred_element_type=jnp.float32),
            0.0) + b1_r[...]
        y = p0 + p1
        mean = jnp.mean(y, axis=0, keepdims=True)
        var = jnp.mean((y - mean) * (y - mean), axis=0, keepdims=True)
        inv = lax.rsqrt(var + EPS) * g_r[...]
        out_r[...] = (y - mean) * inv + be_r[...]

    return pl.pallas_call(
        body,
        out_shape=jax.ShapeDtypeStruct((n, d_out), jnp.float32),
    )(feat, part0, part1, W0, b0.reshape(1, -1), W1, b1.reshape(1, -1),
      gamma.reshape(1, -1), beta.reshape(1, -1))


def kernel(feat, edge_index, edge_weight, W0, b0, W1, b1, gamma, beta):
    n, d = feat.shape
    dst = edge_index[0]
    src = edge_index[1]
    zeros = jnp.zeros((n, d), jnp.float32)
    parts = _sc_spmm_partials(feat, src, dst, edge_weight, zeros)
    return _tc_epilogue(feat, parts[0], parts[1], W0, b0, W1, b1, gamma, beta)


# speedup vs baseline: 4.4395x; 4.4395x over previous
import functools

import jax
import jax.numpy as jnp
from jax import lax
from jax.experimental import pallas as pl
from jax.experimental.pallas import tpu as pltpu
from jax.experimental.pallas import tpu_sc as plsc

EPS = 1e-5

NC = 2
NS = 16
L = 16
NW = NC * NS

K = 80


def _sc_spmm_partials(feat, src, dst, w, zeros):
    n, d = feat.shape
    e = src.shape[0]
    ept = e // NW
    nchunk = ept // K
    rpt = (n // NS) // 8 * 8
    rem = n - NS * rpt
    rem_base = NS * rpt

    mesh = plsc.VectorSubcoreMesh(core_axis_name="c", subcore_axis_name="s")

    @functools.partial(
        pl.kernel,
        mesh=mesh,
        out_type=jax.ShapeDtypeStruct((NC, n, d), jnp.float32),
        scratch_types=[
            pltpu.VMEM((K,), jnp.int32),
            pltpu.VMEM((K,), jnp.int32),
            pltpu.VMEM((K,), jnp.float32),
            pltpu.VMEM((K, d), jnp.float32),
            pltpu.VMEM_SHARED((n, d), jnp.float32),
            pltpu.SemaphoreType.DMA,
        ],
    )
    def k(feat_h, src_h, dst_h, w_h, zeros_h, out_h,
          src_v, dst_v, w_v, rows_v, acc, sem):
        c = lax.axis_index("c")
        s = lax.axis_index("s")
        wid = s * NC + c

        pltpu.sync_copy(zeros_h.at[pl.ds(s * rpt, rpt)],
                        acc.at[pl.ds(s * rpt, rpt)])
        if rem:
            @pl.when(s == 0)
            def _():
                pltpu.sync_copy(zeros_h.at[pl.ds(rem_base, rem)],
                                acc.at[pl.ds(rem_base, rem)])
        plsc.subcore_barrier()

        def body(t, carry):
            base = wid * ept + t * K
            pltpu.sync_copy(src_h.at[pl.ds(base, K)], src_v)
            pltpu.sync_copy(dst_h.at[pl.ds(base, K)], dst_v)
            pltpu.sync_copy(w_h.at[pl.ds(base, K)], w_v)
            pltpu.async_copy(feat_h.at[src_v], rows_v, sem).wait()
            for g in range(K // L):
                wvec = w_v[pl.ds(g * L, L)]
                for ii in range(L):
                    i = g * L + ii
                    ws = wvec[ii]
                    for j in range(d // L):
                        rows_v[i, pl.ds(j * L, L)] = (
                            rows_v[i, pl.ds(j * L, L)] * ws)
            pltpu.sync_copy(rows_v, acc.at[dst_v], add=True)
            return carry

        lax.fori_loop(0, nchunk, body, 0)
        plsc.subcore_barrier()

        pltpu.sync_copy(acc.at[pl.ds(s * rpt, rpt)],
                        out_h.at[c, pl.ds(s * rpt, rpt)])
        if rem:
            @pl.when(s == 0)
            def _():
                pltpu.sync_copy(acc.at[pl.ds(rem_base, rem)],
                                out_h.at[c, pl.ds(rem_base, rem)])

    return k(feat, src, dst, w, zeros)


def _tc_epilogue(feat, part0, part1, W0, b0, W1, b1, gamma, beta):
    n, d_out = feat.shape[0], W0.shape[1]

    def body(feat_r, p0_r, p1_r, w0_r, b0_r, w1_r, b1_r, g_r, be_r, out_r):
        hop1 = p0_r[...] + p1_r[...]
        p0 = jnp.maximum(
            jnp.dot(feat_r[...], w0_r[...], preferred_element_type=jnp.float32),
            0.0) + b0_r[...]
        p1 = jnp.maximum(
            jnp.dot(hop1, w1_r[...], preferred_element_type=jnp.float32),
            0.0) + b1_r[...]
        y = p0 + p1
        mean = jnp.mean(y, axis=0, keepdims=True)
        var = jnp.mean((y - mean) * (y - mean), axis=0, keepdims=True)
        inv = lax.rsqrt(var + EPS) * g_r[...]
        out_r[...] = (y - mean) * inv + be_r[...]

    return pl.pallas_call(
        body,
        out_shape=jax.ShapeDtypeStruct((n, d_out), jnp.float32),
    )(feat, part0, part1, W0, b0.reshape(1, -1), W1, b1.reshape(1, -1),
      gamma.reshape(1, -1), beta.reshape(1, -1))


def kernel(feat, edge_index, edge_weight, W0, b0, W1, b1, gamma, beta):
    n, d = feat.shape
    dst = edge_index[0]
    src = edge_index[1]
    zeros = jnp.zeros((n, d), jnp.float32)
    parts = _sc_spmm_partials(feat, src, dst, edge_weight, zeros)
    return _tc_epilogue(feat, parts[0], parts[1], W0, b0, W1, b1, gamma, beta)


# speedup vs baseline: 10.2812x; 2.3158x over previous
import functools

import jax
import jax.numpy as jnp
from jax import lax
from jax.experimental import pallas as pl
from jax.experimental.pallas import tpu as pltpu
from jax.experimental.pallas import tpu_sc as plsc

EPS = 1e-5

NC = 2
NS = 16
L = 16
NW = NC * NS

K = 80


def _sc_spmm_partials(feat, src, dst, w, zeros):
    n, d = feat.shape
    e = src.shape[0]
    ept = e // NW
    nchunk = ept // K
    rpt = (n // NS) // 8 * 8
    rem = n - NS * rpt
    rem_base = NS * rpt

    mesh = plsc.VectorSubcoreMesh(core_axis_name="c", subcore_axis_name="s")

    @functools.partial(
        pl.kernel,
        mesh=mesh,
        out_type=jax.ShapeDtypeStruct((NC, n, d), jnp.float32),
        scratch_types=[
            pltpu.VMEM((ept,), jnp.int32),
            pltpu.VMEM((K,), jnp.int32),
            pltpu.VMEM((K,), jnp.int32),
            pltpu.VMEM((ept,), jnp.float32),
            pltpu.VMEM((K, d), jnp.float32),
            pltpu.VMEM((K, d), jnp.float32),
            pltpu.VMEM_SHARED((n, d), jnp.float32),
            pltpu.SemaphoreType.DMA,
            pltpu.SemaphoreType.DMA,
            pltpu.SemaphoreType.DMA,
            pltpu.SemaphoreType.DMA,
            pltpu.SemaphoreType.DMA,
        ],
    )
    def k(feat_h, src_h, dst_h, w_h, zeros_h, out_h,
          src_v, dst0, dst1, w_v, rows0, rows1, acc,
          gsem0, gsem1, dsem0, dsem1, ssem):
        c = lax.axis_index("c")
        s = lax.axis_index("s")
        wid = s * NC + c
        rows = (rows0, rows1)
        dstb = (dst0, dst1)
        gsem = (gsem0, gsem1)
        dsem = (dsem0, dsem1)

        def scale_rows(rows_b, t):
            for g in range(K // L):
                wvec = w_v[pl.ds(t * K + g * L, L)]
                for ii in range(L):
                    i = g * L + ii
                    ws = wvec[ii]
                    for j in range(d // L):
                        rows_b[i, pl.ds(j * L, L)] = (
                            rows_b[i, pl.ds(j * L, L)] * ws)

        def issue_gather(t, b):
            pltpu.async_copy(dst_h.at[pl.ds(wid * ept + t * K, K)],
                             dstb[b], dsem[b])
            return pltpu.async_copy(
                feat_h.at[src_v.at[pl.ds(t * K, K)]], rows[b], gsem[b])

        def wait_drain(sem, buf):
            pltpu.make_async_copy(feat_h.at[pl.ds(0, K)], buf, sem).wait()

        def wait_dst(b):
            pltpu.make_async_copy(dst_h.at[pl.ds(0, K)], dstb[b],
                                  dsem[b]).wait()

        def issue_scatter(b):
            pltpu.async_copy(rows[b], acc.at[dstb[b]], ssem, add=True)

        pltpu.sync_copy(src_h.at[pl.ds(wid * ept, ept)], src_v)
        pltpu.sync_copy(w_h.at[pl.ds(wid * ept, ept)], w_v)

        g0 = issue_gather(0, 0)
        issue_gather(1, 1)

        pltpu.sync_copy(zeros_h.at[pl.ds(s * rpt, rpt)],
                        acc.at[pl.ds(s * rpt, rpt)])
        if rem:
            @pl.when(s == 0)
            def _():
                pltpu.sync_copy(zeros_h.at[pl.ds(rem_base, rem)],
                                acc.at[pl.ds(rem_base, rem)])
        plsc.subcore_barrier()

        g0.wait()
        scale_rows(rows0, 0)
        wait_dst(0)
        issue_scatter(0)

        def pair(T, carry):
            t1 = 2 * T + 1
            for b, t in ((1, t1), (0, t1 + 1)):
                o = 1 - b
                wait_drain(ssem, rows[o])
                nxt = t + 1

                @pl.when(nxt < nchunk)
                def _():
                    issue_gather(nxt, o)

                wait_drain(gsem[b], rows[b])
                scale_rows(rows[b], t)
                wait_dst(b)
                issue_scatter(b)
            return carry

        lax.fori_loop(0, (nchunk - 1) // 2, pair, 0)

        wait_drain(ssem, rows0)
        plsc.subcore_barrier()
        pltpu.sync_copy(acc.at[pl.ds(s * rpt, rpt)],
                        out_h.at[c, pl.ds(s * rpt, rpt)])
        if rem:
            @pl.when(s == 0)
            def _():
                pltpu.sync_copy(acc.at[pl.ds(rem_base, rem)],
                                out_h.at[c, pl.ds(rem_base, rem)])

    return k(feat, src, dst, w, zeros)


def _tc_epilogue(feat, part0, part1, W0, b0, W1, b1, gamma, beta):
    n, d_out = feat.shape[0], W0.shape[1]

    def body(feat_r, p0_r, p1_r, w0_r, b0_r, w1_r, b1_r, g_r, be_r, out_r):
        hop1 = p0_r[...] + p1_r[...]
        p0 = jnp.maximum(
            jnp.dot(feat_r[...], w0_r[...], preferred_element_type=jnp.float32),
            0.0) + b0_r[...]
        p1 = jnp.maximum(
            jnp.dot(hop1, w1_r[...], preferred_element_type=jnp.float32),
            0.0) + b1_r[...]
        y = p0 + p1
        mean = jnp.mean(y, axis=0, keepdims=True)
        var = jnp.mean((y - mean) * (y - mean), axis=0, keepdims=True)
        inv = lax.rsqrt(var + EPS) * g_r[...]
        out_r[...] = (y - mean) * inv + be_r[...]

    return pl.pallas_call(
        body,
        out_shape=jax.ShapeDtypeStruct((n, d_out), jnp.float32),
    )(feat, part0, part1, W0, b0.reshape(1, -1), W1, b1.reshape(1, -1),
      gamma.reshape(1, -1), beta.reshape(1, -1))


def kernel(feat, edge_index, edge_weight, W0, b0, W1, b1, gamma, beta):
    n, d = feat.shape
    dst = edge_index[0]
    src = edge_index[1]
    zeros = jnp.zeros((n, d), jnp.float32)
    parts = _sc_spmm_partials(feat, src, dst, edge_weight, zeros)
    return _tc_epilogue(feat, parts[0], parts[1], W0, b0, W1, b1, gamma, beta)
